# baseline (speedup 1.0000x reference)
import functools

import jax
import jax.numpy as jnp
from jax import lax
from jax.experimental import pallas as pl
from jax.experimental.pallas import tpu as pltpu
from jax.experimental.pallas import tpu_sc as plsc

NUM_CLASS = 100000
FEAT_DIM = 128
BATCH = 16384

NUM_CORES = 2
NUM_SUBCORES = 16
LANES = 16
NUM_WORKERS = NUM_CORES * NUM_SUBCORES
ROWS_PER_WORKER = BATCH // NUM_WORKERS
CHUNK = 128
NUM_CHUNKS = ROWS_PER_WORKER // CHUNK
GROUPS_PER_CHUNK = CHUNK // LANES
VECS_PER_ROW = FEAT_DIM // LANES
HALF = FEAT_DIM // 2
SIDE = 128
HALF_ROWS = SIDE // 2
TC_GRID = 16
TC_TILE = SIDE // TC_GRID


def _tree_sum(vals):
    n = len(vals)
    while n > 1:
        vals = [vals[i] + vals[i + 1] for i in range(0, n - 1, 2)] + (
            [vals[-1]] if n % 2 else [])
        n = len(vals)
    return vals[0]


def _sc_body(label_hbm, centers_hbm, b_hbm, g_hbm, idx_v,
             c_v0, c_v1, g_s0, g_s1, b_stage,
             sem_c0, sem_c1, sem_g0, sem_g1):
    wid = lax.axis_index("s") * NUM_CORES + lax.axis_index("c")
    base = pl.multiple_of(wid * ROWS_PER_WORKER, ROWS_PER_WORKER)
    pltpu.sync_copy(label_hbm.at[pl.ds(base, ROWS_PER_WORKER)], idx_v)
    lane_iota = lax.iota(jnp.int32, LANES)
    zero = jnp.zeros((LANES,), jnp.float32)

    c_bufs = (c_v0, c_v1)
    g_stages = (g_s0, g_s1)
    c_sems = (sem_c0, sem_c1)
    g_sems = (sem_g0, sem_g1)

    def gather(k, b):
        return pltpu.make_async_copy(
            centers_hbm.at[idx_v.at[pl.ds(k * CHUNK, CHUNK)]],
            c_bufs[b], c_sems[b])

    def g_write(k, b):
        return pltpu.make_async_copy(
            g_stages[b],
            g_hbm.at[pl.ds(pl.multiple_of((base + k * CHUNK) // 2,
                                          CHUNK // 2), CHUNK // 2)],
            g_sems[b])

    gather(0, 0).start()
    for chunk in range(NUM_CHUNKS):
        b = chunk % 2
        gather(chunk, b).wait()
        if chunk + 1 < NUM_CHUNKS:
            gather(chunk + 1, 1 - b).start()
        if chunk >= 2:
            g_write(chunk - 2, b).wait()
        c_v = c_bufs[b]
        g_stage = g_stages[b]

        def group_body(g, _):
            @plsc.parallel_loop(0, LANES, step=1, unroll=2,
                                carry=(zero, zero))
            def rowloop(r, carry):
                bvl, bvh = carry
                s = g * LANES + r
                cl = [c_v[s, pl.ds(k * LANES, LANES)]
                      for k in range(VECS_PER_ROW)]
                ch = [c_v[s + CHUNK // 2, pl.ds(k * LANES, LANES)]
                      for k in range(VECS_PER_ROW)]
                for k in range(VECS_PER_ROW):
                    packed = plsc.bitcast(
                        plsc.pack(cl[k], ch[k],
                                  format=plsc.PackFormat.INTERLEAVED),
                        jnp.int32)
                    g_stage[s, pl.ds(k * LANES, LANES)] = packed
                cnl = jnp.sum(_tree_sum([cl[k] * cl[k]
                                         for k in range(VECS_PER_ROW)]))
                cnh = jnp.sum(_tree_sum([ch[k] * ch[k]
                                         for k in range(VECS_PER_ROW)]))
                m = lane_iota == r
                return jnp.where(m, cnl, bvl), jnp.where(m, cnh, bvh)

            bvl, bvh = rowloop
            b_stage[pl.ds(chunk * CHUNK + g * LANES, LANES)] = bvl
            b_stage[pl.ds(chunk * CHUNK + CHUNK // 2 + g * LANES,
                          LANES)] = bvh
            return 0

        lax.fori_loop(0, GROUPS_PER_CHUNK // 2, group_body, 0)
        g_write(chunk, b).start()

    g_write(NUM_CHUNKS - 2, 0).wait()
    g_write(NUM_CHUNKS - 1, 1).wait()
    pltpu.sync_copy(b_stage, b_hbm.at[pl.ds(base, ROWS_PER_WORKER)])


@functools.partial(
    pl.kernel,
    out_type=(jax.ShapeDtypeStruct((BATCH,), jnp.float32),
              jax.ShapeDtypeStruct((BATCH // 2, FEAT_DIM), jnp.int32)),
    mesh=plsc.VectorSubcoreMesh(core_axis_name="c", subcore_axis_name="s"),
    compiler_params=pltpu.CompilerParams(needs_layout_passes=False),
    scratch_types=[
        pltpu.VMEM((ROWS_PER_WORKER,), jnp.int32),
        pltpu.VMEM((CHUNK, FEAT_DIM), jnp.float32),
        pltpu.VMEM((CHUNK, FEAT_DIM), jnp.float32),
        pltpu.VMEM((CHUNK // 2, FEAT_DIM), jnp.int32),
        pltpu.VMEM((CHUNK // 2, FEAT_DIM), jnp.int32),
        pltpu.VMEM((ROWS_PER_WORKER,), jnp.float32),
        pltpu.SemaphoreType.DMA,
        pltpu.SemaphoreType.DMA,
        pltpu.SemaphoreType.DMA,
        pltpu.SemaphoreType.DMA,
    ],
)
def _gather_pack(label_hbm, centers_hbm, b_hbm, g_hbm, idx_v,
                 c_v0, c_v1, g_s0, g_s1, b_stage,
                 sem_c0, sem_c1, sem_g0, sem_g1):
    _sc_body(label_hbm, centers_hbm, b_hbm, g_hbm, idx_v,
             c_v0, c_v1, g_s0, g_s1, b_stage,
             sem_c0, sem_c1, sem_g0, sem_g1)


def _fused_tc_kernel(x_ref, g_ref, b_ref, out_ref):
    i = pl.program_id(0)
    xf = x_ref[...]
    g32 = g_ref[...]
    c_lo = jax.lax.bitcast_convert_type(
        jnp.left_shift(g32, 16), jnp.float32)
    c_hi = jax.lax.bitcast_convert_type(
        jnp.bitwise_and(g32, jnp.int32(-65536)), jnp.float32)
    xl = xf[:, :HALF_ROWS, :]
    xh = xf[:, HALF_ROWS:, :]
    dl = jnp.sum(xl * c_lo, axis=-1)
    dh = jnp.sum(xh * c_hi, axis=-1)
    al = jnp.sum(xl * xl, axis=-1)
    ah = jnp.sum(xh * xh, axis=-1)
    bv = b_ref[...]
    bl = bv[:, :HALF_ROWS]
    bh = bv[:, HALF_ROWS:]
    eps = jnp.float32(1e-12)
    den_l = (jnp.maximum(jnp.sqrt(al), eps)
             * jnp.maximum(jnp.sqrt(bl), eps))
    den_h = (jnp.maximum(jnp.sqrt(ah), eps)
             * jnp.maximum(jnp.sqrt(bh), eps))
    cos_l = jnp.clip(dl / den_l, -1.0, 1.0)
    cos_h = jnp.clip(dh / den_h, -1.0, 1.0)
    s = (jnp.sum(cos_l) + jnp.sum(cos_h)) / jnp.float32(BATCH)

    @pl.when(i == 0)
    def _():
        out_ref[...] = jnp.ones((1, 1), jnp.float32)

    out_ref[...] -= jnp.broadcast_to(s, (1, 1))


_fused_tc = pl.pallas_call(
    _fused_tc_kernel,
    grid=(TC_GRID,),
    in_specs=[
        pl.BlockSpec((TC_TILE, SIDE, FEAT_DIM), lambda i: (i, 0, 0)),
        pl.BlockSpec((TC_TILE, SIDE // 2, FEAT_DIM), lambda i: (i, 0, 0)),
        pl.BlockSpec((TC_TILE, SIDE), lambda i: (i, 0)),
    ],
    out_specs=pl.BlockSpec((1, 1), lambda i: (0, 0)),
    out_shape=jax.ShapeDtypeStruct((1, 1), jnp.float32),
)


def kernel(x, label, centers):
    b, g = _gather_pack(label.astype(jnp.int32), centers)
    loss = _fused_tc(x.reshape(SIDE, SIDE, FEAT_DIM),
                     g.reshape(SIDE, SIDE // 2, FEAT_DIM),
                     b.reshape(SIDE, SIDE))
    return loss[0, 0]


# speedup vs baseline: 1.3945x; 1.3945x over previous
import functools

import jax
import jax.numpy as jnp
from jax import lax
from jax.experimental import pallas as pl
from jax.experimental.pallas import tpu as pltpu
from jax.experimental.pallas import tpu_sc as plsc

NUM_CLASS = 100000
FEAT_DIM = 128
BATCH = 16384

NUM_CORES = 2
NUM_SUBCORES = 16
LANES = 16
NUM_WORKERS = NUM_CORES * NUM_SUBCORES
ROWS_PER_WORKER = BATCH // NUM_WORKERS
CHUNK = 128
NUM_CHUNKS = ROWS_PER_WORKER // CHUNK
GROUPS_PER_CHUNK = CHUNK // LANES
VECS_PER_ROW = FEAT_DIM // LANES
SIDE = 128


def _tree_sum(vals):
    n = len(vals)
    while n > 1:
        vals = [vals[i] + vals[i + 1] for i in range(0, n - 1, 2)] + (
            [vals[-1]] if n % 2 else [])
        n = len(vals)
    return vals[0]


def _dot_body(x_hbm, label_hbm, centers_hbm, d_hbm, b_hbm, idx_v,
              x_v0, x_v1, c_v0, c_v1, d_stage, b_stage,
              sem_x0, sem_x1, sem_c0, sem_c1):
    wid = lax.axis_index("s") * NUM_CORES + lax.axis_index("c")
    base = wid * ROWS_PER_WORKER
    pltpu.sync_copy(label_hbm.at[pl.ds(base, ROWS_PER_WORKER)], idx_v)
    lane_iota = lax.iota(jnp.int32, LANES)
    zero = jnp.zeros((LANES,), jnp.float32)

    x_bufs = (x_v0, x_v1)
    c_bufs = (c_v0, c_v1)
    x_sems = (sem_x0, sem_x1)
    c_sems = (sem_c0, sem_c1)

    def copies(k, b):
        dx = pltpu.make_async_copy(
            x_hbm.at[pl.ds((base + k * CHUNK) * FEAT_DIM,
                           CHUNK * FEAT_DIM)], x_bufs[b], x_sems[b])
        dc = pltpu.make_async_copy(
            centers_hbm.at[idx_v.at[pl.ds(k * CHUNK, CHUNK)]],
            c_bufs[b], c_sems[b])
        return dx, dc

    def compute(chunk, x_v, c_v):
        def group_body(g, _):
            @plsc.parallel_loop(0, LANES, step=1, unroll=2,
                                carry=(zero, zero))
            def rowloop(r, carry):
                dvec, bvec = carry
                row = g * LANES + r
                rb = row * FEAT_DIM
                xs = [x_v[pl.ds(rb + k * LANES, LANES)]
                      for k in range(VECS_PER_ROW)]
                cs = [c_v[row, pl.ds(k * LANES, LANES)]
                      for k in range(VECS_PER_ROW)]
                d = jnp.sum(_tree_sum([xs[k] * cs[k]
                                       for k in range(VECS_PER_ROW)]))
                c = jnp.sum(_tree_sum([cs[k] * cs[k]
                                       for k in range(VECS_PER_ROW)]))
                m = lane_iota == r
                return jnp.where(m, d, dvec), jnp.where(m, c, bvec)

            dvec, bvec = rowloop
            off = (chunk * GROUPS_PER_CHUNK + g) * LANES
            d_stage[pl.ds(off, LANES)] = dvec
            b_stage[pl.ds(off, LANES)] = bvec
            return 0

        lax.fori_loop(0, GROUPS_PER_CHUNK, group_body, 0)

    dx, dc = copies(0, 0)
    dx.start()
    dc.start()
    for chunk in range(NUM_CHUNKS):
        b = chunk % 2
        dx, dc = copies(chunk, b)
        dx.wait()
        dc.wait()
        if chunk + 1 < NUM_CHUNKS:
            dx, dc = copies(chunk + 1, 1 - b)
            dx.start()
            dc.start()
        compute(chunk, x_bufs[b], c_bufs[b])

    pltpu.sync_copy(d_stage, d_hbm.at[pl.ds(base, ROWS_PER_WORKER)])
    pltpu.sync_copy(b_stage, b_hbm.at[pl.ds(base, ROWS_PER_WORKER)])


@functools.partial(
    pl.kernel,
    out_type=(jax.ShapeDtypeStruct((BATCH,), jnp.float32),
              jax.ShapeDtypeStruct((BATCH,), jnp.float32)),
    mesh=plsc.VectorSubcoreMesh(core_axis_name="c", subcore_axis_name="s"),
    compiler_params=pltpu.CompilerParams(needs_layout_passes=False),
    scratch_types=[
        pltpu.VMEM((ROWS_PER_WORKER,), jnp.int32),
        pltpu.VMEM((CHUNK * FEAT_DIM,), jnp.float32),
        pltpu.VMEM((CHUNK * FEAT_DIM,), jnp.float32),
        pltpu.VMEM((CHUNK, FEAT_DIM), jnp.float32),
        pltpu.VMEM((CHUNK, FEAT_DIM), jnp.float32),
        pltpu.VMEM((ROWS_PER_WORKER,), jnp.float32),
        pltpu.VMEM((ROWS_PER_WORKER,), jnp.float32),
        pltpu.SemaphoreType.DMA,
        pltpu.SemaphoreType.DMA,
        pltpu.SemaphoreType.DMA,
        pltpu.SemaphoreType.DMA,
    ],
)
def _dot_and_cnorm(x_hbm, label_hbm, centers_hbm, d_hbm, b_hbm, idx_v,
                   x_v0, x_v1, c_v0, c_v1, d_stage, b_stage,
                   sem_x0, sem_x1, sem_c0, sem_c1):
    _dot_body(x_hbm, label_hbm, centers_hbm, d_hbm, b_hbm, idx_v,
              x_v0, x_v1, c_v0, c_v1, d_stage, b_stage,
              sem_x0, sem_x1, sem_c0, sem_c1)


def _xnorm_tc_kernel(x_ref, out_ref):
    x = x_ref[0]
    out_ref[0] = jnp.sum(x * x, axis=1, keepdims=True).T


_xnorm_tc = pl.pallas_call(
    _xnorm_tc_kernel,
    grid=(8,),
    in_specs=[pl.BlockSpec((1, BATCH // 8, FEAT_DIM),
                           lambda i: (i, 0, 0))],
    out_specs=pl.BlockSpec((1, 1, BATCH // 8), lambda i: (i, 0, 0)),
    out_shape=jax.ShapeDtypeStruct((8, 1, BATCH // 8), jnp.float32),
)


def _loss_tc_kernel(d_ref, a_ref, b_ref, out_ref):
    d = d_ref[...]
    a = a_ref[...]
    b = b_ref[...]
    eps = jnp.float32(1e-12)
    denom = (jnp.maximum(jnp.sqrt(a), eps)
             * jnp.maximum(jnp.sqrt(b), eps))
    cos = jnp.clip(d / denom, -1.0, 1.0)
    loss = jnp.float32(1.0) - jnp.sum(cos) / jnp.float32(BATCH)
    out_ref[...] = jnp.broadcast_to(loss, (1, 1))


_loss_tc = pl.pallas_call(
    _loss_tc_kernel,
    out_shape=jax.ShapeDtypeStruct((1, 1), jnp.float32),
)


def kernel(x, label, centers):
    d, b = _dot_and_cnorm(x.reshape(-1), label.astype(jnp.int32), centers)
    a = _xnorm_tc(x.reshape(8, BATCH // 8, FEAT_DIM)).reshape(-1)
    loss = _loss_tc(d.reshape(SIDE, SIDE), a.reshape(SIDE, SIDE),
                    b.reshape(SIDE, SIDE))
    return loss[0, 0]
